# Initial kernel scaffold; baseline (speedup 1.0000x reference)
import jax
import jax.numpy as jnp
from jax.experimental import pallas as pl


def kernel(node_features, edge_index, temporal_info, W1, b1, W2, b2):
    raise NotImplementedError("write your pallas kernel here")


# speedup vs baseline: 3.0028x; 3.0028x over previous
import functools
import jax
import jax.numpy as jnp
from jax.experimental import pallas as pl
from jax.experimental.pallas import tpu as pltpu

N = 10000
E = 320000
D = 128
OUT_DIM = 128
SCALES = 3
EXT = 144
EB = 1280


def _mlp_block(s_ref, d_ref, w1a_ref, w1b_ref, b1_ref, w2_ref, b2_ref, o_ref):
    s = s_ref[...]
    d = d_ref[...]
    u = (
        jax.lax.dot_general(s, w1a_ref[...], (((1,), (0,)), ((), ())),
                            preferred_element_type=jnp.float32)
        + jax.lax.dot_general(d, w1b_ref[...], (((1,), (0,)), ((), ())),
                              preferred_element_type=jnp.float32)
        + b1_ref[...]
    )
    h = jnp.maximum(u, 0.0)
    o_ref[...] = (
        jax.lax.dot_general(h, w2_ref[...], (((1,), (0,)), ((), ())),
                            preferred_element_type=jnp.float32)
        + b2_ref[...]
    )


def _edge_mlp(s_ext, d_ext, w1a, w1b, b1c, w2s, b2m):
    grid = (E // EB,)
    full = lambda shape: pl.BlockSpec(shape, lambda i: tuple(0 for _ in shape))
    return pl.pallas_call(
        _mlp_block,
        grid=grid,
        in_specs=[
            pl.BlockSpec((EB, EXT), lambda i: (i, 0)),
            pl.BlockSpec((EB, EXT), lambda i: (i, 0)),
            full((EXT, SCALES * OUT_DIM)),
            full((EXT, SCALES * OUT_DIM)),
            full((1, SCALES * OUT_DIM)),
            full((SCALES * OUT_DIM, OUT_DIM)),
            full((1, OUT_DIM)),
        ],
        out_specs=pl.BlockSpec((EB, OUT_DIM), lambda i: (i, 0)),
        out_shape=jax.ShapeDtypeStruct((E, OUT_DIM), jnp.float32),
    )(s_ext, d_ext, w1a, w1b, b1c, w2s, b2m)


def kernel(node_features, edge_index, temporal_info, W1, b1, W2, b2):
    w1_src = W1[:, :D, :]
    w1_dst = W1[:, D:2 * D, :]
    w1_t = W1[:, 2 * D, :]
    pad = jnp.zeros((SCALES, EXT - D - 1, OUT_DIM), jnp.float32)
    w1a = jnp.concatenate([w1_src, w1_t[:, None, :], pad], axis=1)
    w1b = jnp.concatenate([w1_dst, -w1_t[:, None, :], pad], axis=1)
    w1a = jnp.transpose(w1a, (1, 0, 2)).reshape(EXT, SCALES * OUT_DIM)
    w1b = jnp.transpose(w1b, (1, 0, 2)).reshape(EXT, SCALES * OUT_DIM)
    b1c = b1.reshape(1, SCALES * OUT_DIM)
    w2s = W2.reshape(SCALES * OUT_DIM, OUT_DIM) / SCALES
    b2m = jnp.mean(b2, axis=0, keepdims=True)

    nf_ext = jnp.concatenate(
        [node_features, temporal_info[:, None],
         jnp.zeros((N, EXT - D - 1), jnp.float32)], axis=1)

    s_ext = jnp.take(nf_ext, edge_index[0], axis=0)
    d_ext = jnp.take(nf_ext, edge_index[1], axis=0)

    return _edge_mlp(s_ext, d_ext, w1a, w1b, b1c, w2s, b2m)


# speedup vs baseline: 3.0411x; 1.0128x over previous
import functools
import jax
import jax.numpy as jnp
from jax.experimental import pallas as pl
from jax.experimental.pallas import tpu as pltpu

N = 10000
E = 320000
D = 128
OUT_DIM = 128
SCALES = 3
EXT = 144
EB = 1280


def _mlp_block(s_ref, d_ref, w1a_ref, w1b_ref, b1_ref, w2_ref, b2_ref, o_ref):
    s = s_ref[...]
    d = d_ref[...]
    u = (
        jax.lax.dot_general(s, w1a_ref[...], (((1,), (0,)), ((), ())),
                            preferred_element_type=jnp.float32)
        + jax.lax.dot_general(d, w1b_ref[...], (((1,), (0,)), ((), ())),
                              preferred_element_type=jnp.float32)
        + b1_ref[...]
    )
    h = jnp.maximum(u, 0.0).astype(jnp.bfloat16)
    o_ref[...] = (
        jax.lax.dot_general(h, w2_ref[...], (((1,), (0,)), ((), ())),
                            preferred_element_type=jnp.float32)
        + b2_ref[...]
    )


def _edge_mlp(s_ext, d_ext, w1a, w1b, b1c, w2s, b2m):
    grid = (E // EB,)
    full = lambda shape: pl.BlockSpec(shape, lambda i: tuple(0 for _ in shape))
    return pl.pallas_call(
        _mlp_block,
        grid=grid,
        in_specs=[
            pl.BlockSpec((EB, EXT), lambda i: (i, 0)),
            pl.BlockSpec((EB, EXT), lambda i: (i, 0)),
            full((EXT, SCALES * OUT_DIM)),
            full((EXT, SCALES * OUT_DIM)),
            full((1, SCALES * OUT_DIM)),
            full((SCALES * OUT_DIM, OUT_DIM)),
            full((1, OUT_DIM)),
        ],
        out_specs=pl.BlockSpec((EB, OUT_DIM), lambda i: (i, 0)),
        out_shape=jax.ShapeDtypeStruct((E, OUT_DIM), jnp.float32),
    )(s_ext, d_ext, w1a, w1b, b1c, w2s, b2m)


def kernel(node_features, edge_index, temporal_info, W1, b1, W2, b2):
    w1_src = W1[:, :D, :]
    w1_dst = W1[:, D:2 * D, :]
    w1_t = W1[:, 2 * D, :]
    pad = jnp.zeros((SCALES, EXT - D - 1, OUT_DIM), jnp.float32)
    w1a = jnp.concatenate([w1_src, w1_t[:, None, :], pad], axis=1)
    w1b = jnp.concatenate([w1_dst, -w1_t[:, None, :], pad], axis=1)
    w1a = jnp.transpose(w1a, (1, 0, 2)).reshape(EXT, SCALES * OUT_DIM)
    w1b = jnp.transpose(w1b, (1, 0, 2)).reshape(EXT, SCALES * OUT_DIM)
    b1c = b1.reshape(1, SCALES * OUT_DIM)
    w2s = (W2.reshape(SCALES * OUT_DIM, OUT_DIM) / SCALES).astype(jnp.bfloat16)
    b2m = jnp.mean(b2, axis=0, keepdims=True)

    nf_ext = jnp.concatenate(
        [node_features, temporal_info[:, None],
         jnp.zeros((N, EXT - D - 1), jnp.float32)], axis=1).astype(jnp.bfloat16)

    s_ext = jnp.take(nf_ext, edge_index[0], axis=0)
    d_ext = jnp.take(nf_ext, edge_index[1], axis=0)

    return _edge_mlp(s_ext, d_ext, w1a.astype(jnp.bfloat16),
                     w1b.astype(jnp.bfloat16), b1c, w2s, b2m)


# speedup vs baseline: 6.9640x; 2.2899x over previous
import functools
import jax
import jax.numpy as jnp
from jax import lax
from jax.experimental import pallas as pl
from jax.experimental.pallas import tpu as pltpu
from jax.experimental.pallas import tpu_sc as plsc

N = 10000
E = 320000
D = 128
OUT_DIM = 128
SCALES = 3
PK = 128
EB = 1280
NW = 32
EPW = E // NW
CH = 80
NCHUNK = EPW // CH


def _sc_gather_body(table, src, dst, s_out, d_out,
                    idx_s, idx_d, rows_s, rows_d, sem_s, sem_d):
    wid = lax.axis_index("s") * 2 + lax.axis_index("c")
    base0 = wid * EPW

    def chunk(i, _):
        base = base0 + i * CH
        pltpu.sync_copy(src.at[pl.ds(base, CH)], idx_s)
        pltpu.sync_copy(dst.at[pl.ds(base, CH)], idx_d)
        cs = pltpu.async_copy(table.at[idx_s], rows_s, sem_s)
        cd = pltpu.async_copy(table.at[idx_d], rows_d, sem_d)
        cs.wait()
        cd.wait()
        pltpu.sync_copy(rows_s, s_out.at[pl.ds(base, CH)])
        pltpu.sync_copy(rows_d, d_out.at[pl.ds(base, CH)])
        return 0

    lax.fori_loop(0, NCHUNK, chunk, 0)


def _sc_gather(table, src, dst):
    mesh = plsc.VectorSubcoreMesh(core_axis_name="c", subcore_axis_name="s")
    fn = functools.partial(
        pl.kernel,
        mesh=mesh,
        out_type=[
            jax.ShapeDtypeStruct((E, PK), jnp.int32),
            jax.ShapeDtypeStruct((E, PK), jnp.int32),
        ],
        scratch_types=[
            pltpu.VMEM((CH,), jnp.int32),
            pltpu.VMEM((CH,), jnp.int32),
            pltpu.VMEM((CH, PK), jnp.int32),
            pltpu.VMEM((CH, PK), jnp.int32),
            pltpu.SemaphoreType.DMA,
            pltpu.SemaphoreType.DMA,
        ],
    )(_sc_gather_body)
    return fn(table, src, dst)


def _unpack_bf16(x):
    lo = lax.bitcast_convert_type(lax.shift_left(x, 16), jnp.float32)
    hi = lax.bitcast_convert_type(
        jnp.bitwise_and(x, jnp.int32(-65536)), jnp.float32)
    return lo.astype(jnp.bfloat16), hi.astype(jnp.bfloat16)


def _mlp_block(s_ref, d_ref, wal_ref, wah_ref, wbl_ref, wbh_ref,
               b1_ref, w2_ref, b2_ref, o_ref):
    mm = lambda a, b: jax.lax.dot_general(
        a, b, (((1,), (0,)), ((), ())), preferred_element_type=jnp.float32)
    s_lo, s_hi = _unpack_bf16(s_ref[...])
    d_lo, d_hi = _unpack_bf16(d_ref[...])
    u = (mm(s_lo, wal_ref[...]) + mm(s_hi, wah_ref[...])
         + mm(d_lo, wbl_ref[...]) + mm(d_hi, wbh_ref[...]) + b1_ref[...])
    h = jnp.maximum(u, 0.0).astype(jnp.bfloat16)
    o_ref[...] = mm(h, w2_ref[...]) + b2_ref[...]


def _edge_mlp(s_pk, d_pk, wal, wah, wbl, wbh, b1c, w2s, b2m):
    grid = (E // EB,)
    full = lambda shape: pl.BlockSpec(shape, lambda i: tuple(0 for _ in shape))
    return pl.pallas_call(
        _mlp_block,
        grid=grid,
        in_specs=[
            pl.BlockSpec((EB, PK), lambda i: (i, 0)),
            pl.BlockSpec((EB, PK), lambda i: (i, 0)),
            full((D, SCALES * OUT_DIM)),
            full((D, SCALES * OUT_DIM)),
            full((D, SCALES * OUT_DIM)),
            full((D, SCALES * OUT_DIM)),
            full((1, SCALES * OUT_DIM)),
            full((SCALES * OUT_DIM, OUT_DIM)),
            full((1, OUT_DIM)),
        ],
        out_specs=pl.BlockSpec((EB, OUT_DIM), lambda i: (i, 0)),
        out_shape=jax.ShapeDtypeStruct((E, OUT_DIM), jnp.float32),
    )(s_pk, d_pk, wal, wah, wbl, wbh, b1c, w2s, b2m)


def kernel(node_features, edge_index, temporal_info, W1, b1, W2, b2):
    f32 = jnp.float32
    w1_src = W1[:, :D, :]
    w1_dst = W1[:, D:2 * D, :]
    w1_t = W1[:, 2 * D, :]
    pad = jnp.zeros((SCALES, 2 * PK - D - 1, OUT_DIM), f32)
    w1a = jnp.concatenate([w1_src, w1_t[:, None, :], pad], axis=1)
    w1b = jnp.concatenate([w1_dst, -w1_t[:, None, :], pad], axis=1)
    w1a = jnp.transpose(w1a, (1, 0, 2)).reshape(2 * PK, SCALES * OUT_DIM)
    w1b = jnp.transpose(w1b, (1, 0, 2)).reshape(2 * PK, SCALES * OUT_DIM)
    bf = jnp.bfloat16
    wal, wah = w1a[0::2].astype(bf), w1a[1::2].astype(bf)
    wbl, wbh = w1b[0::2].astype(bf), w1b[1::2].astype(bf)
    b1c = b1.reshape(1, SCALES * OUT_DIM)
    w2s = (W2.reshape(SCALES * OUT_DIM, OUT_DIM) / SCALES).astype(bf)
    b2m = jnp.mean(b2, axis=0, keepdims=True)

    nf_bf = jnp.concatenate(
        [node_features, temporal_info[:, None],
         jnp.zeros((N, 2 * PK - D - 1), f32)], axis=1).astype(bf)
    table = lax.bitcast_convert_type(nf_bf.reshape(N, PK, 2), jnp.int32)

    eidx = edge_index.astype(jnp.int32)
    s_pk, d_pk = _sc_gather(table, eidx[0], eidx[1])

    return _edge_mlp(s_pk, d_pk, wal, wah, wbl, wbh, b1c, w2s, b2m)


# speedup vs baseline: 10.2138x; 1.4667x over previous
import functools
import jax
import jax.numpy as jnp
from jax import lax
from jax.experimental import pallas as pl
from jax.experimental.pallas import tpu as pltpu
from jax.experimental.pallas import tpu_sc as plsc

N = 10000
E = 320000
D = 128
OUT_DIM = 128
SCALES = 3
PK = 128
EB = 1280
NW = 32
EPW = E // NW
CH = 40
K = 5
GE = K * CH
NG = EPW // GE
NPAIR = NG // 2


def _sc_gather_body(table, src, dst, s_out, d_out,
                    idx_s, idx_d, rows, sem_g0, sem_g1, sem_w0, sem_w1):
    wid = lax.axis_index("s") * 2 + lax.axis_index("c")
    base0 = wid * EPW
    sem_g = (sem_g0, sem_g1)
    sem_w = (sem_w0, sem_w1)

    pltpu.sync_copy(src.at[pl.ds(base0, EPW)], idx_s)
    pltpu.sync_copy(dst.at[pl.ds(base0, EPW)], idx_d)

    def gathers(st, g):
        ops = []
        for k in range(K):
            off = g * GE + k * CH
            ops.append(pltpu.async_copy(
                table.at[idx_s.at[pl.ds(off, CH)]], rows.at[st, 2 * k],
                sem_g[st]))
            ops.append(pltpu.async_copy(
                table.at[idx_d.at[pl.ds(off, CH)]], rows.at[st, 2 * k + 1],
                sem_g[st]))
        return ops

    def drain_g(st):
        for k in range(2 * K):
            pltpu.make_async_copy(
                table.at[idx_s.at[pl.ds(0, CH)]], rows.at[st, k],
                sem_g[st]).wait()

    def writes(st, g):
        for k in range(K):
            off = g * GE + k * CH
            pltpu.async_copy(rows.at[st, 2 * k],
                             s_out.at[pl.ds(base0 + off, CH)], sem_w[st])
            pltpu.async_copy(rows.at[st, 2 * k + 1],
                             d_out.at[pl.ds(base0 + off, CH)], sem_w[st])

    def drain_w(st, g):
        for k in range(K):
            off = g * GE + k * CH
            pltpu.make_async_copy(
                rows.at[st, 2 * k], s_out.at[pl.ds(base0 + off, CH)],
                sem_w[st]).wait()
            pltpu.make_async_copy(
                rows.at[st, 2 * k + 1], d_out.at[pl.ds(base0 + off, CH)],
                sem_w[st]).wait()

    gathers(0, 0)

    def body(j, _):
        g0 = 2 * j

        @pl.when(j > 0)
        def _():
            drain_w(1, g0 - 1)

        gathers(1, g0 + 1)
        drain_g(0)
        writes(0, g0)
        drain_w(0, g0)

        @pl.when(j < NPAIR - 1)
        def _():
            gathers(0, g0 + 2)

        drain_g(1)
        writes(1, g0 + 1)
        return 0

    lax.fori_loop(0, NPAIR, body, 0)
    drain_w(1, NG - 1)


def _sc_gather(table, src, dst):
    mesh = plsc.VectorSubcoreMesh(core_axis_name="c", subcore_axis_name="s")
    fn = functools.partial(
        pl.kernel,
        mesh=mesh,
        out_type=[
            jax.ShapeDtypeStruct((E, PK), jnp.int32),
            jax.ShapeDtypeStruct((E, PK), jnp.int32),
        ],
        scratch_types=[
            pltpu.VMEM((EPW,), jnp.int32),
            pltpu.VMEM((EPW,), jnp.int32),
            pltpu.VMEM((2, 2 * K, CH, PK), jnp.int32),
            pltpu.SemaphoreType.DMA,
            pltpu.SemaphoreType.DMA,
            pltpu.SemaphoreType.DMA,
            pltpu.SemaphoreType.DMA,
        ],
    )(_sc_gather_body)
    return fn(table, src, dst)


def _mlp_block(s_ref, d_ref, wa_ref, wb_ref, wt_ref, b1_ref, w2_ref, b2_ref,
               o_ref):
    mm = lambda a, b: jax.lax.dot_general(
        a, b, (((1,), (0,)), ((), ())), preferred_element_type=jnp.float32)
    bf = jnp.bfloat16
    s = s_ref[...]
    d = d_ref[...]
    s_lo = lax.bitcast_convert_type(lax.shift_left(s, 16), jnp.float32)
    d_lo = lax.bitcast_convert_type(lax.shift_left(d, 16), jnp.float32)
    mask = jnp.int32(-65536)
    gap = (lax.bitcast_convert_type(jnp.bitwise_and(s, mask), jnp.float32)
           - lax.bitcast_convert_type(jnp.bitwise_and(d, mask), jnp.float32))
    u = (mm(s_lo.astype(bf), wa_ref[...]) + mm(d_lo.astype(bf), wb_ref[...])
         + mm(gap.astype(bf), wt_ref[...]) + b1_ref[...])
    h = jnp.maximum(u, 0.0).astype(bf)
    o_ref[...] = mm(h, w2_ref[...]) + b2_ref[...]


def _edge_mlp(s_pk, d_pk, wa, wb, wt, b1c, w2s, b2m):
    grid = (E // EB,)
    full = lambda shape: pl.BlockSpec(shape, lambda i: tuple(0 for _ in shape))
    return pl.pallas_call(
        _mlp_block,
        grid=grid,
        in_specs=[
            pl.BlockSpec((EB, PK), lambda i: (i, 0)),
            pl.BlockSpec((EB, PK), lambda i: (i, 0)),
            full((D, SCALES * OUT_DIM)),
            full((D, SCALES * OUT_DIM)),
            full((D, SCALES * OUT_DIM)),
            full((1, SCALES * OUT_DIM)),
            full((SCALES * OUT_DIM, OUT_DIM)),
            full((1, OUT_DIM)),
        ],
        out_specs=pl.BlockSpec((EB, OUT_DIM), lambda i: (i, 0)),
        out_shape=jax.ShapeDtypeStruct((E, OUT_DIM), jnp.float32),
    )(s_pk, d_pk, wa, wb, wt, b1c, w2s, b2m)


def kernel(node_features, edge_index, temporal_info, W1, b1, W2, b2):
    f32, bf, i32 = jnp.float32, jnp.bfloat16, jnp.int32
    cat = lambda w: jnp.transpose(w, (1, 0, 2)).reshape(
        w.shape[1], SCALES * OUT_DIM)
    wa = cat(W1[:, :D, :]).astype(bf)
    wb = cat(W1[:, D:2 * D, :]).astype(bf)
    w_t = cat(W1[:, 2 * D:2 * D + 1, :])
    wt = jnp.concatenate([w_t, jnp.zeros((D - 1, SCALES * OUT_DIM), f32)],
                         axis=0).astype(bf)
    b1c = b1.reshape(1, SCALES * OUT_DIM)
    w2s = (W2.reshape(SCALES * OUT_DIM, OUT_DIM) / SCALES).astype(bf)
    b2m = jnp.mean(b2, axis=0, keepdims=True)

    lo = lax.bitcast_convert_type(node_features.astype(bf), jnp.uint16)
    lo = lo.astype(i32)
    t_bits = lax.bitcast_convert_type(temporal_info.astype(bf), jnp.uint16)
    hi = jnp.concatenate(
        [t_bits.astype(i32)[:, None], jnp.zeros((N, D - 1), i32)], axis=1)
    table = jnp.bitwise_or(lo, lax.shift_left(hi, 16))

    eidx = edge_index.astype(i32)
    s_pk, d_pk = _sc_gather(table, eidx[0], eidx[1])

    return _edge_mlp(s_pk, d_pk, wa, wb, wt, b1c, w2s, b2m)


# speedup vs baseline: 12.8067x; 1.2539x over previous
import functools
import jax
import jax.numpy as jnp
from jax import lax
from jax.experimental import pallas as pl
from jax.experimental.pallas import tpu as pltpu
from jax.experimental.pallas import tpu_sc as plsc

N = 10000
E = 320000
D = 128
OUT_DIM = 128
SCALES = 3
PK = 128
EB = 2560
NW = 32
EPW = E // NW
CH = 40
K = 5
GE = K * CH
NG = EPW // GE
NPAIR = NG // 2


def _sc_gather_body(table, src, dst, s_out, d_out,
                    idx_s, idx_d, rows, sem_g0, sem_g1, sem_w0, sem_w1):
    wid = lax.axis_index("s") * 2 + lax.axis_index("c")
    base0 = wid * EPW
    sem_g = (sem_g0, sem_g1)
    sem_w = (sem_w0, sem_w1)

    pltpu.sync_copy(src.at[pl.ds(base0, EPW)], idx_s)
    pltpu.sync_copy(dst.at[pl.ds(base0, EPW)], idx_d)

    def gathers(st, g):
        ops = []
        for k in range(K):
            off = g * GE + k * CH
            ops.append(pltpu.async_copy(
                table.at[idx_s.at[pl.ds(off, CH)]], rows.at[st, 2 * k],
                sem_g[st]))
            ops.append(pltpu.async_copy(
                table.at[idx_d.at[pl.ds(off, CH)]], rows.at[st, 2 * k + 1],
                sem_g[st]))
        return ops

    def drain_g(st):
        for k in range(2 * K):
            pltpu.make_async_copy(
                table.at[idx_s.at[pl.ds(0, CH)]], rows.at[st, k],
                sem_g[st]).wait()

    def writes(st, g):
        for k in range(K):
            off = g * GE + k * CH
            pltpu.async_copy(rows.at[st, 2 * k],
                             s_out.at[pl.ds(base0 + off, CH)], sem_w[st])
            pltpu.async_copy(rows.at[st, 2 * k + 1],
                             d_out.at[pl.ds(base0 + off, CH)], sem_w[st])

    def drain_w(st, g):
        for k in range(K):
            off = g * GE + k * CH
            pltpu.make_async_copy(
                rows.at[st, 2 * k], s_out.at[pl.ds(base0 + off, CH)],
                sem_w[st]).wait()
            pltpu.make_async_copy(
                rows.at[st, 2 * k + 1], d_out.at[pl.ds(base0 + off, CH)],
                sem_w[st]).wait()

    gathers(0, 0)

    def body(j, _):
        g0 = 2 * j

        @pl.when(j > 0)
        def _():
            drain_w(1, g0 - 1)

        gathers(1, g0 + 1)
        drain_g(0)
        writes(0, g0)
        drain_w(0, g0)

        @pl.when(j < NPAIR - 1)
        def _():
            gathers(0, g0 + 2)

        drain_g(1)
        writes(1, g0 + 1)
        return 0

    lax.fori_loop(0, NPAIR, body, 0)
    drain_w(1, NG - 1)


def _sc_gather(table, src, dst):
    mesh = plsc.VectorSubcoreMesh(core_axis_name="c", subcore_axis_name="s")
    fn = functools.partial(
        pl.kernel,
        mesh=mesh,
        out_type=[
            jax.ShapeDtypeStruct((E, PK), jnp.int32),
            jax.ShapeDtypeStruct((E, PK), jnp.int32),
        ],
        scratch_types=[
            pltpu.VMEM((EPW,), jnp.int32),
            pltpu.VMEM((EPW,), jnp.int32),
            pltpu.VMEM((2, 2 * K, CH, PK), jnp.int32),
            pltpu.SemaphoreType.DMA,
            pltpu.SemaphoreType.DMA,
            pltpu.SemaphoreType.DMA,
            pltpu.SemaphoreType.DMA,
        ],
    )(_sc_gather_body)
    return fn(table, src, dst)


def _mlp_block(s_ref, d_ref, w1_ref, b1_ref, w2_ref, b2_ref, o_ref):
    mm = lambda a, b: jax.lax.dot_general(
        a, b, (((1,), (0,)), ((), ())), preferred_element_type=jnp.float32)
    bf = jnp.bfloat16
    s = s_ref[...]
    d = d_ref[...]
    s_lo = lax.bitcast_convert_type(lax.shift_left(s, 16), jnp.float32)
    d_lo = lax.bitcast_convert_type(lax.shift_left(d, 16), jnp.float32)
    gap = (lax.bitcast_convert_type(s, jnp.float32)
           - lax.bitcast_convert_type(d, jnp.float32))
    x = jnp.concatenate(
        [s_lo.astype(bf), d_lo.astype(bf), gap.astype(bf)], axis=1)
    u = mm(x, w1_ref[...]) + b1_ref[...]
    h = jnp.maximum(u, 0.0).astype(bf)
    o_ref[...] = mm(h, w2_ref[...]) + b2_ref[...]


def _edge_mlp(s_pk, d_pk, w1c, b1c, w2s, b2m):
    grid = (E // EB,)
    full = lambda shape: pl.BlockSpec(shape, lambda i: tuple(0 for _ in shape))
    return pl.pallas_call(
        _mlp_block,
        grid=grid,
        in_specs=[
            pl.BlockSpec((EB, PK), lambda i: (i, 0)),
            pl.BlockSpec((EB, PK), lambda i: (i, 0)),
            full((3 * D, SCALES * OUT_DIM)),
            full((1, SCALES * OUT_DIM)),
            full((SCALES * OUT_DIM, OUT_DIM)),
            full((1, OUT_DIM)),
        ],
        out_specs=pl.BlockSpec((EB, OUT_DIM), lambda i: (i, 0)),
        out_shape=jax.ShapeDtypeStruct((E, OUT_DIM), jnp.float32),
    )(s_pk, d_pk, w1c, b1c, w2s, b2m)


def kernel(node_features, edge_index, temporal_info, W1, b1, W2, b2):
    f32, bf, i32 = jnp.float32, jnp.bfloat16, jnp.int32
    cat = lambda w: jnp.transpose(w, (1, 0, 2)).reshape(
        w.shape[1], SCALES * OUT_DIM)
    wa = cat(W1[:, :D, :])
    wb = cat(W1[:, D:2 * D, :])
    w_t = cat(W1[:, 2 * D:2 * D + 1, :])
    wt = jnp.concatenate([w_t, jnp.zeros((D - 1, SCALES * OUT_DIM), f32)],
                         axis=0)
    w1c = jnp.concatenate([wa, wb, wt], axis=0).astype(bf)
    b1c = b1.reshape(1, SCALES * OUT_DIM)
    w2s = (W2.reshape(SCALES * OUT_DIM, OUT_DIM) / SCALES).astype(bf)
    b2m = jnp.mean(b2, axis=0, keepdims=True)

    lo = lax.bitcast_convert_type(node_features.astype(bf), jnp.uint16)
    lo = lo.astype(i32)
    t_bits = lax.bitcast_convert_type(temporal_info.astype(bf), jnp.uint16)
    hi = jnp.concatenate(
        [t_bits.astype(i32)[:, None], jnp.zeros((N, D - 1), i32)], axis=1)
    table = jnp.bitwise_or(lo, lax.shift_left(hi, 16))

    eidx = edge_index.astype(i32)
    s_pk, d_pk = _sc_gather(table, eidx[0], eidx[1])

    return _edge_mlp(s_pk, d_pk, w1c, b1c, w2s, b2m)
